# baseline (speedup 1.0000x reference)
import functools

import jax
import jax.numpy as jnp
from jax import lax
from jax.experimental import pallas as pl
from jax.experimental.pallas import tpu as pltpu
from jax.experimental.pallas import tpu_sc as plsc

NUM_EMBEDDINGS = 1000000
EMBEDDING_DIM = 64
BATCH = 16384
HIST_LEN = 20
B_TOTAL = BATCH * HIST_LEN

NUM_CORES = 2
NUM_SUBCORES = 16
NUM_WORKERS = NUM_CORES * NUM_SUBCORES
B_PER_W = B_TOTAL // NUM_WORKERS
CHUNK = 256
N_CHUNKS = B_PER_W // CHUNK
NBUF = 6
LOOK = 4

@functools.lru_cache(maxsize=1)
def _build_embedding_gather():
    mesh = plsc.VectorSubcoreMesh(core_axis_name="c", subcore_axis_name="s")

    @functools.partial(
        pl.kernel,
        mesh=mesh,
        compiler_params=pltpu.CompilerParams(use_tc_tiling_on_sc=False),
        out_type=jax.ShapeDtypeStruct((B_TOTAL, EMBEDDING_DIM), jnp.float32),
        scratch_types=(
            [pltpu.VMEM((B_PER_W,), jnp.int32)]
            + [pltpu.VMEM((CHUNK, EMBEDDING_DIM), jnp.float32)] * NBUF
            + [pltpu.SemaphoreType.DMA] * (2 * NBUF)
        ),
    )
    def _embedding_gather(table_hbm, idx_hbm, out_hbm, idx_v, *scratch):
        bufs = scratch[:NBUF]
        gsems = scratch[NBUF : 2 * NBUF]
        wsems = scratch[2 * NBUF :]
        wid = lax.axis_index("s") * NUM_CORES + lax.axis_index("c")
        base = wid * B_PER_W
        pltpu.sync_copy(idx_hbm.at[pl.ds(base, B_PER_W)], idx_v)

        def gather(c, b):
            return pltpu.async_copy(
                table_hbm.at[idx_v.at[pl.ds(c * CHUNK, CHUNK)]], bufs[b], gsems[b]
            )

        gh = [None] * NBUF
        wh = [None] * NBUF
        for g in range(min(LOOK, N_CHUNKS)):
            gh[g % NBUF] = gather(g, g % NBUF)
        for c in range(N_CHUNKS):
            g = c + LOOK
            if g < N_CHUNKS:
                b = g % NBUF
                if wh[b] is not None:
                    wh[b].wait()
                    wh[b] = None
                gh[b] = gather(g, b)
            b = c % NBUF
            gh[b].wait()
            wh[b] = pltpu.async_copy(
                bufs[b], out_hbm.at[pl.ds(base + c * CHUNK, CHUNK)], wsems[b]
            )
        for b in range(NBUF):
            if wh[b] is not None:
                wh[b].wait()

    return _embedding_gather


def kernel(input_, weight):
    idx = input_.reshape((B_TOTAL,)).astype(jnp.int32)
    out = _build_embedding_gather()(weight, idx)
    return out.reshape((BATCH, HIST_LEN, EMBEDDING_DIM))


# speedup vs baseline: 2.0539x; 2.0539x over previous
import functools

import jax
import jax.numpy as jnp
from jax import lax
from jax.experimental import pallas as pl
from jax.experimental.pallas import tpu as pltpu
from jax.experimental.pallas import tpu_sc as plsc

NUM_EMBEDDINGS = 1000000
EMBEDDING_DIM = 64
BATCH = 16384
HIST_LEN = 20
B_TOTAL = BATCH * HIST_LEN

NUM_CORES = 2
NUM_SUBCORES = 16
J_PER_CORE = EMBEDDING_DIM // NUM_CORES
B_PER_TILE = BATCH // NUM_SUBCORES
ELEMS_PER_TILE = HIST_LEN * B_PER_TILE


@functools.lru_cache(maxsize=1)
def _build_planf():
    mesh = plsc.VectorSubcoreMesh(core_axis_name="c", subcore_axis_name="s")

    @functools.partial(
        pl.kernel,
        mesh=mesh,
        compiler_params=pltpu.CompilerParams(use_tc_tiling_on_sc=True),
        out_type=jax.ShapeDtypeStruct((HIST_LEN, EMBEDDING_DIM, BATCH), jnp.float32),
        scratch_types=[
            pltpu.VMEM_SHARED((NUM_EMBEDDINGS,), jnp.float32),
            pltpu.VMEM((HIST_LEN * B_PER_TILE,), jnp.int32),
            pltpu.VMEM((HIST_LEN * B_PER_TILE,), jnp.float32),
            pltpu.VMEM((HIST_LEN * B_PER_TILE,), jnp.float32),
            pltpu.SemaphoreType.DMA,
            pltpu.SemaphoreType.DMA,
            pltpu.SemaphoreType.DMA,
            pltpu.SemaphoreType.DMA,
        ],
    )
    def _planf(wt_hbm, idx_hbm, out_hbm, row_sp, idx_v, val0, val1, rsem, gsem0, gsem1, wsem):
        c = lax.axis_index("c")
        s = lax.axis_index("s")
        for h in range(HIST_LEN):
            pltpu.sync_copy(
                idx_hbm.at[pl.ds(h * BATCH + s * B_PER_TILE, B_PER_TILE)],
                idx_v.at[pl.ds(h * B_PER_TILE, B_PER_TILE)],
            )

        vals = (val0, val1)
        gsems = (gsem0, gsem1)

        def load_row(jj):
            j = c * J_PER_CORE + jj
            return pltpu.async_copy(wt_hbm.at[j], row_sp, rsem)

        def gather(buf_i):
            return pltpu.async_copy(row_sp.at[idx_v], vals[buf_i], gsems[buf_i])

        def write_out(jj, buf_i):
            j = c * J_PER_CORE + jj
            hs = []
            for h in range(HIST_LEN):
                hs.append(
                    pltpu.async_copy(
                        vals[buf_i].at[pl.ds(h * B_PER_TILE, B_PER_TILE)],
                        out_hbm.at[h, j, pl.ds(s * B_PER_TILE, B_PER_TILE)],
                        wsem,
                    )
                )
            return hs

        for jj in range(J_PER_CORE):
            buf_i = jj % 2
            @pl.when(s == 0)
            def _():
                load_row(jj).wait()
            plsc.subcore_barrier()
            gather(buf_i).wait()
            plsc.subcore_barrier()
            for hnd in write_out(jj, buf_i):
                hnd.wait()

    return _planf


def kernel(input_, weight):
    wt = weight.T
    idxt = input_.T.astype(jnp.int32).reshape((B_TOTAL,))
    out_t = _build_planf()(wt, idxt)
    return out_t.transpose(2, 0, 1)


# speedup vs baseline: 2.1573x; 1.0503x over previous
import functools

import jax
import jax.numpy as jnp
from jax import lax
from jax.experimental import pallas as pl
from jax.experimental.pallas import tpu as pltpu
from jax.experimental.pallas import tpu_sc as plsc

NUM_EMBEDDINGS = 1000000
EMBEDDING_DIM = 64
BATCH = 16384
HIST_LEN = 20
B_TOTAL = BATCH * HIST_LEN

NUM_CORES = 2
NUM_SUBCORES = 16
J_PER_CORE = EMBEDDING_DIM // NUM_CORES
B_PER_TILE = BATCH // NUM_SUBCORES
ELEMS_PER_TILE = HIST_LEN * B_PER_TILE
N_GSTREAMS = 4
G_CHUNK = ELEMS_PER_TILE // N_GSTREAMS


@functools.lru_cache(maxsize=1)
def _build_planf():
    mesh = plsc.VectorSubcoreMesh(core_axis_name="c", subcore_axis_name="s")

    @functools.partial(
        pl.kernel,
        mesh=mesh,
        compiler_params=pltpu.CompilerParams(use_tc_tiling_on_sc=True),
        out_type=jax.ShapeDtypeStruct((HIST_LEN, EMBEDDING_DIM, BATCH), jnp.float32),
        scratch_types=(
            [pltpu.VMEM_SHARED((NUM_EMBEDDINGS,), jnp.float32)]
            + [pltpu.VMEM((ELEMS_PER_TILE,), jnp.int32)]
            + [pltpu.VMEM((ELEMS_PER_TILE,), jnp.float32)] * 2
            + [pltpu.SemaphoreType.DMA] * (1 + N_GSTREAMS + 2)
        ),
    )
    def _planf(wt_hbm, idx_hbm, out_hbm, row_sp, idx_v, val0, val1, *sems):
        rsem = sems[0]
        gsems = sems[1 : 1 + N_GSTREAMS]
        wsems = sems[1 + N_GSTREAMS :]
        c = lax.axis_index("c")
        s = lax.axis_index("s")
        for h in range(HIST_LEN):
            pltpu.sync_copy(
                idx_hbm.at[pl.ds(h * BATCH + s * B_PER_TILE, B_PER_TILE)],
                idx_v.at[pl.ds(h * B_PER_TILE, B_PER_TILE)],
            )

        vals = (val0, val1)

        def row_copy(jj):
            j = c * J_PER_CORE + jj
            return pltpu.make_async_copy(wt_hbm.at[j], row_sp, rsem)

        def gather(jj):
            hs = []
            for g in range(N_GSTREAMS):
                hs.append(
                    pltpu.async_copy(
                        row_sp.at[idx_v.at[pl.ds(g * G_CHUNK, G_CHUNK)]],
                        vals[jj % 2].at[pl.ds(g * G_CHUNK, G_CHUNK)],
                        gsems[g],
                    )
                )
            return hs

        def write_out(jj):
            j = c * J_PER_CORE + jj
            hs = []
            for h in range(HIST_LEN):
                hs.append(
                    pltpu.async_copy(
                        vals[jj % 2].at[pl.ds(h * B_PER_TILE, B_PER_TILE)],
                        out_hbm.at[h, j, pl.ds(s * B_PER_TILE, B_PER_TILE)],
                        wsems[jj % 2],
                    )
                )
            return hs

        write_handles = [None] * J_PER_CORE

        first_copy = row_copy(0)

        @pl.when(s == 0)
        def _():
            first_copy.start()

        for jj in range(J_PER_CORE):
            cur_copy = row_copy(jj)

            @pl.when(s == 0)
            def _():
                cur_copy.wait()
            if jj >= 2:
                for hnd in write_handles[jj - 2]:
                    hnd.wait()
            plsc.subcore_barrier()
            for hnd in gather(jj):
                hnd.wait()
            plsc.subcore_barrier()

            if jj + 1 < J_PER_CORE:
                next_copy = row_copy(jj + 1)

                @pl.when(s == 0)
                def _():
                    next_copy.start()

            write_handles[jj] = write_out(jj)

        for jj in (J_PER_CORE - 2, J_PER_CORE - 1):
            for hnd in write_handles[jj]:
                hnd.wait()

    return _planf


def kernel(input_, weight):
    wt = weight.T
    idxt = input_.T.astype(jnp.int32).reshape((B_TOTAL,))
    out_t = _build_planf()(wt, idxt)
    return out_t.transpose(2, 0, 1)


# speedup vs baseline: 2.1590x; 1.0008x over previous
import functools

import jax
import jax.numpy as jnp
from jax import lax
from jax.experimental import pallas as pl
from jax.experimental.pallas import tpu as pltpu
from jax.experimental.pallas import tpu_sc as plsc

NUM_EMBEDDINGS = 1000000
EMBEDDING_DIM = 64
BATCH = 16384
HIST_LEN = 20
B_TOTAL = BATCH * HIST_LEN

NUM_CORES = 2
NUM_SUBCORES = 16
J_PER_CORE = EMBEDDING_DIM // NUM_CORES
B_PER_TILE = BATCH // NUM_SUBCORES
ELEMS_PER_TILE = HIST_LEN * B_PER_TILE
N_GSTREAMS = 4
G_CHUNK = ELEMS_PER_TILE // N_GSTREAMS


@functools.lru_cache(maxsize=1)
def _build_planf():
    mesh = plsc.VectorSubcoreMesh(core_axis_name="c", subcore_axis_name="s")

    @functools.partial(
        pl.kernel,
        mesh=mesh,
        compiler_params=pltpu.CompilerParams(use_tc_tiling_on_sc=True),
        out_type=jax.ShapeDtypeStruct((HIST_LEN, EMBEDDING_DIM, BATCH), jnp.float32),
        scratch_types=(
            [pltpu.VMEM_SHARED((NUM_EMBEDDINGS,), jnp.float32)]
            + [pltpu.VMEM((ELEMS_PER_TILE,), jnp.int32)]
            + [pltpu.VMEM((ELEMS_PER_TILE,), jnp.float32)] * 2
            + [pltpu.SemaphoreType.DMA] * (1 + N_GSTREAMS + 2)
        ),
    )
    def _planf(wt_hbm, idx_hbm, out_hbm, row_sp, idx_v, val0, val1, *sems):
        rsem = sems[0]
        gsems = sems[1 : 1 + N_GSTREAMS]
        wsems = sems[1 + N_GSTREAMS :]
        c = lax.axis_index("c")
        s = lax.axis_index("s")
        for h in range(HIST_LEN):
            pltpu.sync_copy(
                idx_hbm.at[pl.ds(h * BATCH + s * B_PER_TILE, B_PER_TILE)],
                idx_v.at[pl.ds(h * B_PER_TILE, B_PER_TILE)],
            )

        vals = (val0, val1)

        def row_copy(jj):
            j = c * J_PER_CORE + jj
            return pltpu.make_async_copy(wt_hbm.at[j], row_sp, rsem)

        H_PER_STREAM = HIST_LEN // N_GSTREAMS

        def gather(jj):
            hs = []
            for g in range(N_GSTREAMS):
                hs.append(
                    pltpu.async_copy(
                        row_sp.at[idx_v.at[pl.ds(g * G_CHUNK, G_CHUNK)]],
                        vals[jj % 2].at[pl.ds(g * G_CHUNK, G_CHUNK)],
                        gsems[g],
                    )
                )
            return hs

        def write_h(jj, h):
            j = c * J_PER_CORE + jj
            return pltpu.async_copy(
                vals[jj % 2].at[pl.ds(h * B_PER_TILE, B_PER_TILE)],
                out_hbm.at[h, j, pl.ds(s * B_PER_TILE, B_PER_TILE)],
                wsems[jj % 2],
            )

        write_handles = [None] * J_PER_CORE

        first_copy = row_copy(0)

        @pl.when(s == 0)
        def _():
            first_copy.start()

        for jj in range(J_PER_CORE):
            cur_copy = row_copy(jj)

            @pl.when(s == 0)
            def _():
                cur_copy.wait()
            if jj >= 2:
                for hnd in write_handles[jj - 2]:
                    hnd.wait()
            plsc.subcore_barrier()
            ghs = gather(jj)
            whs = []
            for g in range(N_GSTREAMS):
                ghs[g].wait()
                for h in range(g * H_PER_STREAM, (g + 1) * H_PER_STREAM):
                    whs.append(write_h(jj, h))
            plsc.subcore_barrier()

            if jj + 1 < J_PER_CORE:
                next_copy = row_copy(jj + 1)

                @pl.when(s == 0)
                def _():
                    next_copy.start()

            write_handles[jj] = whs

        for jj in (J_PER_CORE - 2, J_PER_CORE - 1):
            for hnd in write_handles[jj]:
                hnd.wait()

    return _planf


def kernel(input_, weight):
    wt = weight.T
    idxt = input_.T.astype(jnp.int32).reshape((B_TOTAL,))
    out_t = _build_planf()(wt, idxt)
    return out_t.transpose(2, 0, 1)


# speedup vs baseline: 2.2175x; 1.0271x over previous
import functools

import jax
import jax.numpy as jnp
from jax import lax
from jax.experimental import pallas as pl
from jax.experimental.pallas import tpu as pltpu
from jax.experimental.pallas import tpu_sc as plsc

NUM_EMBEDDINGS = 1000000
EMBEDDING_DIM = 64
BATCH = 16384
HIST_LEN = 20
B_TOTAL = BATCH * HIST_LEN

NUM_CORES = 2
NUM_SUBCORES = 16
J_PER_CORE = EMBEDDING_DIM // NUM_CORES
B_PER_TILE = BATCH // NUM_SUBCORES
ELEMS_PER_TILE = HIST_LEN * B_PER_TILE
N_GSTREAMS = 4
G_CHUNK = ELEMS_PER_TILE // N_GSTREAMS


@functools.lru_cache(maxsize=1)
def _build_planf():
    mesh = plsc.VectorSubcoreMesh(core_axis_name="c", subcore_axis_name="s")

    @functools.partial(
        pl.kernel,
        mesh=mesh,
        compiler_params=pltpu.CompilerParams(use_tc_tiling_on_sc=True),
        out_type=jax.ShapeDtypeStruct((HIST_LEN, EMBEDDING_DIM, BATCH), jnp.float32),
        scratch_types=(
            [pltpu.VMEM_SHARED((NUM_EMBEDDINGS,), jnp.float32)]
            + [pltpu.VMEM((ELEMS_PER_TILE,), jnp.int32)]
            + [pltpu.VMEM((ELEMS_PER_TILE,), jnp.float32)] * 2
            + [pltpu.SemaphoreType.DMA] * (1 + N_GSTREAMS + 2)
        ),
    )
    def _planf(wt_hbm, idx_hbm, out_hbm, row_sp, idx_v, val0, val1, *sems):
        rsem = sems[0]
        gsems = sems[1 : 1 + N_GSTREAMS]
        wsems = sems[1 + N_GSTREAMS :]
        c = lax.axis_index("c")
        s = lax.axis_index("s")
        vals = (val0, val1)

        def row_copy(jj):
            j = c * J_PER_CORE + jj
            return pltpu.make_async_copy(wt_hbm.at[j], row_sp, rsem)

        H_PER_STREAM = HIST_LEN // N_GSTREAMS

        def gather(jj):
            hs = []
            for g in range(N_GSTREAMS):
                hs.append(
                    pltpu.async_copy(
                        row_sp.at[idx_v.at[pl.ds(g * G_CHUNK, G_CHUNK)]],
                        vals[jj % 2].at[pl.ds(g * G_CHUNK, G_CHUNK)],
                        gsems[g],
                    )
                )
            return hs

        def write_h(jj, h):
            j = c * J_PER_CORE + jj
            return pltpu.async_copy(
                vals[jj % 2].at[pl.ds(h * B_PER_TILE, B_PER_TILE)],
                out_hbm.at[h, j, pl.ds(s * B_PER_TILE, B_PER_TILE)],
                wsems[jj % 2],
            )

        write_handles = [None] * J_PER_CORE

        first_copy = row_copy(0)

        @pl.when(s == 0)
        def _():
            first_copy.start()

        stage = [
            pltpu.make_async_copy(
                idx_hbm.at[pl.ds(h * BATCH + s * B_PER_TILE, B_PER_TILE)],
                idx_v.at[pl.ds(h * B_PER_TILE, B_PER_TILE)],
                wsems[0],
            )
            for h in range(HIST_LEN)
        ]
        for cp in stage:
            cp.start()
        for cp in stage:
            cp.wait()

        for jj in range(J_PER_CORE):
            cur_copy = row_copy(jj)

            @pl.when(s == 0)
            def _():
                cur_copy.wait()
            if jj >= 2:
                for hnd in write_handles[jj - 2]:
                    hnd.wait()
            plsc.subcore_barrier()
            ghs = gather(jj)
            whs = []
            for g in range(N_GSTREAMS):
                ghs[g].wait()
                for h in range(g * H_PER_STREAM, (g + 1) * H_PER_STREAM):
                    whs.append(write_h(jj, h))
            plsc.subcore_barrier()

            if jj + 1 < J_PER_CORE:
                next_copy = row_copy(jj + 1)

                @pl.when(s == 0)
                def _():
                    next_copy.start()

            write_handles[jj] = whs

        for jj in (J_PER_CORE - 2, J_PER_CORE - 1):
            for hnd in write_handles[jj]:
                hnd.wait()

    return _planf


def kernel(input_, weight):
    wt = weight.T
    idxt = input_.T.astype(jnp.int32).reshape((B_TOTAL,))
    out_t = _build_planf()(wt, idxt)
    return out_t.transpose(2, 0, 1)
